# baseline (speedup 1.0000x reference)
import functools

import jax
import jax.numpy as jnp
from jax import lax
from jax.experimental import pallas as pl
from jax.experimental.pallas import tpu as pltpu
from jax.experimental.pallas import tpu_sc as plsc

N = 10000
D_SLOTS = 16
T_ROUNDS = 3
F = 70
FP = 128
NP = 10240
EP = 8

_info = plsc.get_sparse_core_info()
_NC, _NS = _info.num_cores, _info.num_subcores
_NW = _NC * _NS
_BPW = NP // _NW

_NCHUNK = 4
_CH = _BPW // _NCHUNK


def _sc_gather_body(j, table_hbm, idx_hbm, out_hbm, idx_v, rows_v, sem_g, sem_w):
    wid = lax.axis_index("s") * _NC + lax.axis_index("c")
    base = wid * _BPW
    pltpu.sync_copy(idx_hbm.at[pl.ds(j * NP + base, _BPW)], idx_v)
    pltpu.async_copy(table_hbm.at[idx_v], rows_v, sem_g).wait()
    pltpu.async_copy(rows_v, out_hbm.at[pl.ds(base, _BPW)], sem_w).wait()


def _make_sc_gather(j):
    return pl.kernel(
        functools.partial(_sc_gather_body, j),
        out_type=jax.ShapeDtypeStruct((NP, FP), jnp.float32),
        mesh=plsc.VectorSubcoreMesh(core_axis_name="c", subcore_axis_name="s"),
        scratch_types=[
            pltpu.VMEM((_BPW,), jnp.int32),
            pltpu.VMEM((_BPW, FP), jnp.float32),
            pltpu.SemaphoreType.DMA,
            pltpu.SemaphoreType.DMA,
        ],
    )


_sc_gathers = [_make_sc_gather(j) for j in range(D_SLOTS)]


def _tc_step_body(h_ref, g_ref, e_ref, wv_ref, bv_ref, wes_ref, bes_ref,
                  wu_ref, bu_ref, out_ref):
    m = jnp.dot(g_ref[...], wv_ref[...], preferred_element_type=jnp.float32) + bv_ref[...]
    e = e_ref[0].astype(jnp.float32)
    wes = wes_ref[...].astype(jnp.float32)
    for k in range(6):
        m = m + e[:, k:k + 1] * wes[k:k + 1, :]
    m = m + bes_ref[...]
    cat = jnp.concatenate([h_ref[...], m], axis=1)
    z = jnp.dot(cat, wu_ref[...], preferred_element_type=jnp.float32)
    out_ref[...] = jnp.maximum(z + bu_ref[...], 0.0)


def _make_tc_step(j):
    return pl.pallas_call(
        _tc_step_body,
        out_shape=jax.ShapeDtypeStruct((NP, FP), jnp.float32),
        grid=(1,),
        in_specs=[
            pl.BlockSpec((NP, FP), lambda i: (0, 0)),
            pl.BlockSpec((NP, FP), lambda i: (0, 0)),
            pl.BlockSpec((1, NP, EP), lambda i: (j, 0, 0)),
            pl.BlockSpec((FP, FP), lambda i: (0, 0)),
            pl.BlockSpec((1, FP), lambda i: (0, 0)),
            pl.BlockSpec((EP, FP), lambda i: (0, 0)),
            pl.BlockSpec((1, FP), lambda i: (0, 0)),
            pl.BlockSpec((2 * FP, FP), lambda i: (0, 0)),
            pl.BlockSpec((1, FP), lambda i: (0, 0)),
        ],
        out_specs=pl.BlockSpec((NP, FP), lambda i: (0, 0)),
    )


_tc_steps = [_make_tc_step(j) for j in range(D_SLOTS)]


def _tc_readout_body(h_ref, x_ref, wrh_ref, wrx_ref, br_ref, out_ref):
    z = jnp.dot(h_ref[...], wrh_ref[...], preferred_element_type=jnp.float32)
    z = z + jnp.dot(x_ref[...], wrx_ref[...], preferred_element_type=jnp.float32)
    z = jnp.maximum(z + br_ref[...], 0.0)
    rows = lax.broadcasted_iota(jnp.int32, (NP, 128), 0)
    z = jnp.where(rows < N, z, 0.0)
    out_ref[...] = jnp.sum(z, axis=0, keepdims=True)


_tc_readout = pl.pallas_call(
    _tc_readout_body,
    out_shape=jax.ShapeDtypeStruct((1, 128), jnp.float32),
)


def kernel(x, neighbors, edge_attr, W_R, b_R, W_U, b_U, W_V, b_V, W_E, b_E,
           W_s1, b_s1, W_s2, b_s2, W_h, b_h, W_o, b_o):
    f32 = jnp.float32

    wv = jnp.zeros((FP, FP), f32).at[:F, :F].set(W_V.T)
    bv = jnp.zeros((1, FP), f32).at[0, :F].set(b_V)
    wes = jnp.zeros((EP, FP), f32).at[:6, F:F + 6].set(W_E.T)
    bes = jnp.zeros((1, FP), f32).at[0, F:F + 6].set(b_E)
    wu = (jnp.zeros((2 * FP, FP), f32)
          .at[:F, :F].set(W_U[:, :F].T)
          .at[FP:FP + F, :F].set(W_U[:, F:2 * F].T)
          .at[FP + F:FP + F + 6, :F].set(W_U[:, 2 * F:].T))
    bu = jnp.zeros((1, FP), f32).at[0, :F].set(b_U)

    x_pad = jnp.zeros((NP, FP), f32).at[:N, :F].set(x)
    idx_all = jnp.zeros((D_SLOTS, NP), jnp.int32).at[:, :N].set(
        neighbors.astype(jnp.int32).T).reshape(D_SLOTS * NP)
    e_all = jnp.zeros((D_SLOTS, NP, EP), f32).at[:, :N, :6].set(
        jnp.transpose(edge_attr, (1, 0, 2)))

    wrh = jnp.zeros((FP, 128), f32).at[:F, :].set(W_R[:, :F].T)
    wrx = jnp.zeros((FP, 128), f32).at[:F, :].set(W_R[:, F:].T)
    br = b_R.reshape(1, 128)

    h = x_pad
    for _ in range(T_ROUNDS):
        for j in range(D_SLOTS):
            g = _sc_gathers[j](h, idx_all)
            h = _tc_steps[j](h, g, e_all, wv, bv, wes, bes, wu, bu)

    fm = _tc_readout(h, x_pad, wrh, wrx, br)[0]
    shared = jax.nn.relu(jax.nn.relu(fm @ W_s1.T + b_s1)) @ W_s2.T + b_s2
    hidden = jax.nn.relu(shared @ W_h.T + b_h)
    return jax.nn.relu(hidden) @ W_o.T + b_o


# speedup vs baseline: 1.0202x; 1.0202x over previous
import functools

import jax
import jax.numpy as jnp
from jax import lax
from jax.experimental import pallas as pl
from jax.experimental.pallas import tpu as pltpu
from jax.experimental.pallas import tpu_sc as plsc

N = 10000
D_SLOTS = 16
T_ROUNDS = 3
F = 70
FP = 128
NP = 10240
EP = 8

_info = plsc.get_sparse_core_info()
_NC, _NS = _info.num_cores, _info.num_subcores
_NW = _NC * _NS
_BPW = NP // _NW

_NCHUNK = 4
_CH = _BPW // _NCHUNK


def _sc_gather_body(j, table_hbm, idx_hbm, out_hbm, idx_v, rows_v, sem_g, sem_w):
    wid = lax.axis_index("s") * _NC + lax.axis_index("c")
    base = wid * _BPW
    pltpu.sync_copy(idx_hbm.at[pl.ds(j * NP + base, _BPW)], idx_v)
    gathers = []
    for c in range(_NCHUNK):
        gathers.append(pltpu.async_copy(
            table_hbm.at[idx_v.at[pl.ds(c * _CH, _CH)]],
            rows_v.at[pl.ds(c * _CH, _CH)], sem_g))
    writes = []
    for c in range(_NCHUNK):
        gathers[c].wait()
        writes.append(pltpu.async_copy(
            rows_v.at[pl.ds(c * _CH, _CH)],
            out_hbm.at[pl.ds(base + c * _CH, _CH)], sem_w))
    for w in writes:
        w.wait()


def _make_sc_gather(j):
    return pl.kernel(
        functools.partial(_sc_gather_body, j),
        out_type=jax.ShapeDtypeStruct((NP, FP), jnp.float32),
        mesh=plsc.VectorSubcoreMesh(core_axis_name="c", subcore_axis_name="s"),
        scratch_types=[
            pltpu.VMEM((_BPW,), jnp.int32),
            pltpu.VMEM((_BPW, FP), jnp.float32),
            pltpu.SemaphoreType.DMA,
            pltpu.SemaphoreType.DMA,
        ],
    )


_sc_gathers = [_make_sc_gather(j) for j in range(D_SLOTS)]


def _tc_step_body(h_ref, g_ref, e_ref, wv_ref, bv_ref, wes_ref, bes_ref,
                  wu_ref, bu_ref, out_ref):
    m = jnp.dot(g_ref[...], wv_ref[...], preferred_element_type=jnp.float32) + bv_ref[...]
    e = e_ref[0].astype(jnp.float32)
    wes = wes_ref[...].astype(jnp.float32)
    for k in range(6):
        m = m + e[:, k:k + 1] * wes[k:k + 1, :]
    m = m + bes_ref[...]
    cat = jnp.concatenate([h_ref[...], m], axis=1)
    z = jnp.dot(cat, wu_ref[...], preferred_element_type=jnp.float32)
    out_ref[...] = jnp.maximum(z + bu_ref[...], 0.0)


def _make_tc_step(j):
    return pl.pallas_call(
        _tc_step_body,
        out_shape=jax.ShapeDtypeStruct((NP, FP), jnp.float32),
        grid=(1,),
        in_specs=[
            pl.BlockSpec((NP, FP), lambda i: (0, 0)),
            pl.BlockSpec((NP, FP), lambda i: (0, 0)),
            pl.BlockSpec((1, NP, EP), lambda i: (j, 0, 0)),
            pl.BlockSpec((FP, FP), lambda i: (0, 0)),
            pl.BlockSpec((1, FP), lambda i: (0, 0)),
            pl.BlockSpec((EP, FP), lambda i: (0, 0)),
            pl.BlockSpec((1, FP), lambda i: (0, 0)),
            pl.BlockSpec((2 * FP, FP), lambda i: (0, 0)),
            pl.BlockSpec((1, FP), lambda i: (0, 0)),
        ],
        out_specs=pl.BlockSpec((NP, FP), lambda i: (0, 0)),
    )


_tc_steps = [_make_tc_step(j) for j in range(D_SLOTS)]


def _tc_readout_body(h_ref, x_ref, wrh_ref, wrx_ref, br_ref, out_ref):
    z = jnp.dot(h_ref[...], wrh_ref[...], preferred_element_type=jnp.float32)
    z = z + jnp.dot(x_ref[...], wrx_ref[...], preferred_element_type=jnp.float32)
    z = jnp.maximum(z + br_ref[...], 0.0)
    rows = lax.broadcasted_iota(jnp.int32, (NP, 128), 0)
    z = jnp.where(rows < N, z, 0.0)
    out_ref[...] = jnp.sum(z, axis=0, keepdims=True)


_tc_readout = pl.pallas_call(
    _tc_readout_body,
    out_shape=jax.ShapeDtypeStruct((1, 128), jnp.float32),
)


def kernel(x, neighbors, edge_attr, W_R, b_R, W_U, b_U, W_V, b_V, W_E, b_E,
           W_s1, b_s1, W_s2, b_s2, W_h, b_h, W_o, b_o):
    f32 = jnp.float32

    wv = jnp.zeros((FP, FP), f32).at[:F, :F].set(W_V.T)
    bv = jnp.zeros((1, FP), f32).at[0, :F].set(b_V)
    wes = jnp.zeros((EP, FP), f32).at[:6, F:F + 6].set(W_E.T)
    bes = jnp.zeros((1, FP), f32).at[0, F:F + 6].set(b_E)
    wu = (jnp.zeros((2 * FP, FP), f32)
          .at[:F, :F].set(W_U[:, :F].T)
          .at[FP:FP + F, :F].set(W_U[:, F:2 * F].T)
          .at[FP + F:FP + F + 6, :F].set(W_U[:, 2 * F:].T))
    bu = jnp.zeros((1, FP), f32).at[0, :F].set(b_U)

    x_pad = jnp.zeros((NP, FP), f32).at[:N, :F].set(x)
    idx_all = jnp.zeros((D_SLOTS, NP), jnp.int32).at[:, :N].set(
        neighbors.astype(jnp.int32).T).reshape(D_SLOTS * NP)
    e_all = jnp.zeros((D_SLOTS, NP, EP), f32).at[:, :N, :6].set(
        jnp.transpose(edge_attr, (1, 0, 2)))

    wrh = jnp.zeros((FP, 128), f32).at[:F, :].set(W_R[:, :F].T)
    wrx = jnp.zeros((FP, 128), f32).at[:F, :].set(W_R[:, F:].T)
    br = b_R.reshape(1, 128)

    h = x_pad
    for _ in range(T_ROUNDS):
        for j in range(D_SLOTS):
            g = _sc_gathers[j](h, idx_all)
            h = _tc_steps[j](h, g, e_all, wv, bv, wes, bes, wu, bu)

    fm = _tc_readout(h, x_pad, wrh, wrx, br)[0]
    shared = jax.nn.relu(jax.nn.relu(fm @ W_s1.T + b_s1)) @ W_s2.T + b_s2
    hidden = jax.nn.relu(shared @ W_h.T + b_h)
    return jax.nn.relu(hidden) @ W_o.T + b_o


# speedup vs baseline: 1.1035x; 1.0817x over previous
import functools

import jax
import jax.numpy as jnp
from jax import lax
from jax.experimental import pallas as pl
from jax.experimental.pallas import tpu as pltpu
from jax.experimental.pallas import tpu_sc as plsc

N = 10000
D_SLOTS = 16
T_ROUNDS = 3
F = 70
FP = 128
NP = 10240
EP = 8

_info = plsc.get_sparse_core_info()
_NC, _NS = _info.num_cores, _info.num_subcores
_NW = _NC * _NS
_BPW = NP // _NW

_NCHUNK = 4
_CH = _BPW // _NCHUNK


def _sc_gather_body(j, table_hbm, idx_hbm, out_hbm, idx_v, rows_v, sem_g, sem_w):
    wid = lax.axis_index("s") * _NC + lax.axis_index("c")
    base = wid * _BPW
    pltpu.sync_copy(idx_hbm.at[pl.ds(j * NP + base, _BPW)], idx_v)
    gathers = []
    for c in range(_NCHUNK):
        gathers.append(pltpu.async_copy(
            table_hbm.at[idx_v.at[pl.ds(c * _CH, _CH)]],
            rows_v.at[pl.ds(c * _CH, _CH)], sem_g))
    writes = []
    for c in range(_NCHUNK):
        gathers[c].wait()
        writes.append(pltpu.async_copy(
            rows_v.at[pl.ds(c * _CH, _CH)],
            out_hbm.at[pl.ds(base + c * _CH, _CH)], sem_w))
    for w in writes:
        w.wait()


def _make_sc_gather(j):
    return pl.kernel(
        functools.partial(_sc_gather_body, j),
        out_type=jax.ShapeDtypeStruct((NP, FP), jnp.float32),
        mesh=plsc.VectorSubcoreMesh(core_axis_name="c", subcore_axis_name="s"),
        scratch_types=[
            pltpu.VMEM((_BPW,), jnp.int32),
            pltpu.VMEM((_BPW, FP), jnp.float32),
            pltpu.SemaphoreType.DMA,
            pltpu.SemaphoreType.DMA,
        ],
    )


_sc_gathers = [_make_sc_gather(j) for j in range(D_SLOTS)]


def _tc_step_body(h_ref, g_ref, e_ref, wv_ref, bv_ref, wes_ref, bes_ref,
                  wu_ref, bu_ref, out_ref):
    m = jnp.dot(g_ref[...], wv_ref[...], preferred_element_type=jnp.float32) + bv_ref[...]
    e = e_ref[0].astype(jnp.float32)
    wes = wes_ref[...].astype(jnp.float32)
    for k in range(6):
        m = m + e[:, k:k + 1] * wes[k:k + 1, :]
    m = m + bes_ref[...]
    cat = jnp.concatenate([h_ref[...], m], axis=1)
    z = jnp.dot(cat, wu_ref[...], preferred_element_type=jnp.float32)
    out_ref[...] = jnp.maximum(z + bu_ref[...], 0.0)


_MT = 8
_MB = NP // _MT


def _make_tc_step(j):
    return pl.pallas_call(
        _tc_step_body,
        out_shape=jax.ShapeDtypeStruct((NP, FP), jnp.float32),
        grid=(_MT,),
        in_specs=[
            pl.BlockSpec((_MB, FP), lambda i: (i, 0)),
            pl.BlockSpec((_MB, FP), lambda i: (i, 0)),
            pl.BlockSpec((1, _MB, EP), lambda i: (j, i, 0)),
            pl.BlockSpec((FP, FP), lambda i: (0, 0)),
            pl.BlockSpec((1, FP), lambda i: (0, 0)),
            pl.BlockSpec((EP, FP), lambda i: (0, 0)),
            pl.BlockSpec((1, FP), lambda i: (0, 0)),
            pl.BlockSpec((2 * FP, FP), lambda i: (0, 0)),
            pl.BlockSpec((1, FP), lambda i: (0, 0)),
        ],
        out_specs=pl.BlockSpec((_MB, FP), lambda i: (i, 0)),
    )


_tc_steps = [_make_tc_step(j) for j in range(D_SLOTS)]


def _tc_readout_body(h_ref, x_ref, wrh_ref, wrx_ref, br_ref, out_ref):
    z = jnp.dot(h_ref[...], wrh_ref[...], preferred_element_type=jnp.float32)
    z = z + jnp.dot(x_ref[...], wrx_ref[...], preferred_element_type=jnp.float32)
    z = jnp.maximum(z + br_ref[...], 0.0)
    rows = lax.broadcasted_iota(jnp.int32, (NP, 128), 0)
    z = jnp.where(rows < N, z, 0.0)
    out_ref[...] = jnp.sum(z, axis=0, keepdims=True)


_tc_readout = pl.pallas_call(
    _tc_readout_body,
    out_shape=jax.ShapeDtypeStruct((1, 128), jnp.float32),
)


def kernel(x, neighbors, edge_attr, W_R, b_R, W_U, b_U, W_V, b_V, W_E, b_E,
           W_s1, b_s1, W_s2, b_s2, W_h, b_h, W_o, b_o):
    f32 = jnp.float32

    wv = jnp.zeros((FP, FP), f32).at[:F, :F].set(W_V.T)
    bv = jnp.zeros((1, FP), f32).at[0, :F].set(b_V)
    wes = jnp.zeros((EP, FP), f32).at[:6, F:F + 6].set(W_E.T)
    bes = jnp.zeros((1, FP), f32).at[0, F:F + 6].set(b_E)
    wu = (jnp.zeros((2 * FP, FP), f32)
          .at[:F, :F].set(W_U[:, :F].T)
          .at[FP:FP + F, :F].set(W_U[:, F:2 * F].T)
          .at[FP + F:FP + F + 6, :F].set(W_U[:, 2 * F:].T))
    bu = jnp.zeros((1, FP), f32).at[0, :F].set(b_U)

    x_pad = jnp.zeros((NP, FP), f32).at[:N, :F].set(x)
    idx_all = jnp.zeros((D_SLOTS, NP), jnp.int32).at[:, :N].set(
        neighbors.astype(jnp.int32).T).reshape(D_SLOTS * NP)
    e_all = jnp.zeros((D_SLOTS, NP, EP), f32).at[:, :N, :6].set(
        jnp.transpose(edge_attr, (1, 0, 2)))

    wrh = jnp.zeros((FP, 128), f32).at[:F, :].set(W_R[:, :F].T)
    wrx = jnp.zeros((FP, 128), f32).at[:F, :].set(W_R[:, F:].T)
    br = b_R.reshape(1, 128)

    h = x_pad
    for _ in range(T_ROUNDS):
        for j in range(D_SLOTS):
            g = _sc_gathers[j](h, idx_all)
            h = _tc_steps[j](h, g, e_all, wv, bv, wes, bes, wu, bu)

    fm = _tc_readout(h, x_pad, wrh, wrx, br)[0]
    shared = jax.nn.relu(jax.nn.relu(fm @ W_s1.T + b_s1)) @ W_s2.T + b_s2
    hidden = jax.nn.relu(shared @ W_h.T + b_h)
    return jax.nn.relu(hidden) @ W_o.T + b_o
